# baseline (speedup 1.0000x reference)
import functools

import jax
import jax.numpy as jnp
from jax import lax
from jax.experimental import pallas as pl
from jax.experimental.pallas import tpu as pltpu, tpu_sc as plsc

N = 10000
E = 320000
D = 128
P = 100000

N_PAD = 10240
P_PAD = 102400

NC, NS = 2, 16
NW = NC * NS

E_PAD = 327680
E_PER_W = E_PAD // NW
E_CHUNK = 256
E_STEPS = E_PER_W // E_CHUNK

DEG_CHUNK = 512
DEG_STEPS = E_PER_W // DEG_CHUNK

P_PER_W = P_PAD // NW
P_CHUNK = 400
P_STEPS = P_PER_W // P_CHUNK

ROWS_PER_SUB = N_PAD // NS

_mesh = plsc.VectorSubcoreMesh(core_axis_name="c", subcore_axis_name="s")


@functools.partial(
    pl.kernel,
    out_type=jax.ShapeDtypeStruct((NC, N_PAD), jnp.float32),
    mesh=_mesh,
    scratch_types=[
        pltpu.VMEM((E_PER_W,), jnp.int32),
        pltpu.VMEM((DEG_CHUNK,), jnp.float32),
        pltpu.VMEM_SHARED((N_PAD,), jnp.float32),
    ],
)
def _sc_degree(dst_hbm, zeros_hbm, out_hbm, didx_v, ones_v, acc_sh):
    cid = lax.axis_index("c")
    sid = lax.axis_index("s")
    wid = sid * NC + cid

    def fill(i, carry):
        ones_v[pl.ds(i * 16, 16)] = jnp.full((16,), 1.0, jnp.float32)
        return carry
    lax.fori_loop(0, DEG_CHUNK // 16, fill, None)

    e0 = pl.multiple_of(wid * E_PER_W, 8)
    pltpu.sync_copy(dst_hbm.at[pl.ds(e0, E_PER_W)], didx_v)

    row0 = sid * ROWS_PER_SUB
    pltpu.sync_copy(zeros_hbm.at[pl.ds(row0, ROWS_PER_SUB)],
                    acc_sh.at[pl.ds(row0, ROWS_PER_SUB)])
    plsc.subcore_barrier()

    def step(j, carry):
        idx = didx_v.at[pl.ds(j * DEG_CHUNK, DEG_CHUNK)]
        pltpu.sync_copy(ones_v, acc_sh.at[idx], add=True)
        return carry
    lax.fori_loop(0, DEG_STEPS, step, None)

    plsc.subcore_barrier()
    pltpu.sync_copy(acc_sh.at[pl.ds(row0, ROWS_PER_SUB)],
                    out_hbm.at[cid, pl.ds(row0, ROWS_PER_SUB)])


@functools.partial(
    pl.kernel,
    out_type=jax.ShapeDtypeStruct((NC, N_PAD, D), jnp.float32),
    mesh=_mesh,
    scratch_types=[
        pltpu.VMEM((E_CHUNK,), jnp.int32),
        pltpu.VMEM((E_CHUNK,), jnp.int32),
        pltpu.VMEM((E_CHUNK, D), jnp.float32),
        pltpu.VMEM_SHARED((N_PAD, D), jnp.float32),
        pltpu.SemaphoreType.DMA,
    ],
)
def _sc_message_pass(hp_hbm, src_hbm, dst_hbm, zeros_hbm, out_hbm,
                     sidx_v, didx_v, rows0, acc_sh, sem0):
    cid = lax.axis_index("c")
    sid = lax.axis_index("s")
    wid = sid * NC + cid

    row0 = sid * ROWS_PER_SUB
    @pl.when(cid == 0)
    def _():
        pltpu.sync_copy(hp_hbm.at[pl.ds(row0, ROWS_PER_SUB)],
                        acc_sh.at[pl.ds(row0, ROWS_PER_SUB)])
    @pl.when(cid != 0)
    def _():
        pltpu.sync_copy(zeros_hbm.at[pl.ds(row0, ROWS_PER_SUB)],
                        acc_sh.at[pl.ds(row0, ROWS_PER_SUB)])
    plsc.subcore_barrier()

    def step(t, carry):
        e0 = pl.multiple_of(wid * E_PER_W + t * E_CHUNK, 8)
        pltpu.sync_copy(src_hbm.at[pl.ds(e0, E_CHUNK)], sidx_v)
        pltpu.sync_copy(dst_hbm.at[pl.ds(e0, E_CHUNK)], didx_v)
        pltpu.async_copy(hp_hbm.at[sidx_v], rows0, sem0).wait()
        pltpu.sync_copy(rows0, acc_sh.at[didx_v], add=True)
        return carry
    lax.fori_loop(0, E_STEPS, step, None)

    plsc.subcore_barrier()
    pltpu.sync_copy(acc_sh.at[pl.ds(row0, ROWS_PER_SUB)],
                    out_hbm.at[cid, pl.ds(row0, ROWS_PER_SUB)])


@functools.partial(
    pl.kernel,
    out_type=(jax.ShapeDtypeStruct((P_PAD, D), jnp.float32),
              jax.ShapeDtypeStruct((P_PAD, D), jnp.float32)),
    mesh=_mesh,
    scratch_types=[
        pltpu.VMEM((P_CHUNK,), jnp.int32),
        pltpu.VMEM((P_CHUNK,), jnp.int32),
        pltpu.VMEM((P_CHUNK, D), jnp.float32),
        pltpu.VMEM((P_CHUNK, D), jnp.float32),
        pltpu.SemaphoreType.DMA,
    ],
)
def _sc_pair_gather(a_hbm, b_hbm, sidx_hbm, tidx_hbm, ga_hbm, gb_hbm,
                    si_v, ti_v, a0, b0, sem0):
    cid = lax.axis_index("c")
    sid = lax.axis_index("s")
    wid = sid * NC + cid

    def step(t, carry):
        p0 = pl.multiple_of(wid * P_PER_W + t * P_CHUNK, 8)
        pltpu.sync_copy(sidx_hbm.at[pl.ds(p0, P_CHUNK)], si_v)
        pltpu.sync_copy(tidx_hbm.at[pl.ds(p0, P_CHUNK)], ti_v)
        cpa = pltpu.async_copy(a_hbm.at[si_v], a0, sem0)
        cpb = pltpu.async_copy(b_hbm.at[ti_v], b0, sem0)
        cpa.wait()
        cpb.wait()
        pltpu.sync_copy(a0, ga_hbm.at[pl.ds(p0, P_CHUNK)])
        pltpu.sync_copy(b0, gb_hbm.at[pl.ds(p0, P_CHUNK)])
        return carry
    lax.fori_loop(0, P_STEPS, step, None)


ROW_BLK = 512
N_GRID = N_PAD // ROW_BLK

_HIGH = jax.lax.Precision.HIGHEST


def _dinv_from(deg_ref):
    deg = deg_ref[0] + deg_ref[1] + 1.0
    return lax.rsqrt(deg)


def _tc1_body(x_ref, w1_ref, wlb_ref, bl1_ref, deg_ref, h1p_ref, bpre_ref):
    x = x_ref[...]
    dinv = _dinv_from(deg_ref)
    h = jnp.dot(x, w1_ref[...], precision=_HIGH, preferred_element_type=jnp.float32)
    h1p_ref[...] = h * dinv
    bpre_ref[...] = jnp.dot(x, wlb_ref[...], precision=_HIGH,
                            preferred_element_type=jnp.float32) + bl1_ref[...]


def _tc_mid_body(relu_first, s_ref, deg_ref, b_ref, w_ref, out_ref):
    dinv = _dinv_from(deg_ref)
    s = s_ref[0] + s_ref[1]
    h = dinv * s + b_ref[...]
    if relu_first:
        h = jnp.maximum(h, 0.0)
        out_ref[...] = jnp.dot(h, w_ref[...], precision=_HIGH,
                               preferred_element_type=jnp.float32) * dinv
    else:
        out_ref[...] = jnp.dot(h, w_ref[...], precision=_HIGH,
                               preferred_element_type=jnp.float32)


PAIR_BLK = 2048
P_GRID = P_PAD // PAIR_BLK


def _tc4_body(ga_ref, gb_ref, wl2_ref, bl2_ref, out_ref):
    z1 = jnp.maximum(ga_ref[...] + gb_ref[...], 0.0)
    z = jnp.dot(z1, wl2_ref[...], precision=_HIGH,
                preferred_element_type=jnp.float32) + bl2_ref[...]
    out_ref[...] = jax.nn.sigmoid(z)


def _row_spec(blk, cols):
    return pl.BlockSpec((blk, cols), lambda i: (i, 0))


def _full_spec(shape):
    return pl.BlockSpec(shape, lambda i: tuple(0 for _ in shape))


def _tc1(x_pad, W1, Wlb, bl1, deg2):
    return pl.pallas_call(
        _tc1_body,
        grid=(N_GRID,),
        in_specs=[
            _row_spec(ROW_BLK, D),
            _full_spec((D, D)),
            _full_spec((D, D)),
            _full_spec((1, D)),
            pl.BlockSpec((NC, ROW_BLK, 1), lambda i: (0, i, 0)),
        ],
        out_specs=[_row_spec(ROW_BLK, D), _row_spec(ROW_BLK, D)],
        out_shape=[jax.ShapeDtypeStruct((N_PAD, D), jnp.float32),
                   jax.ShapeDtypeStruct((N_PAD, D), jnp.float32)],
    )(x_pad, W1, Wlb, bl1, deg2)


def _tc_mid(relu_first, S, deg2, b, W):
    return pl.pallas_call(
        functools.partial(_tc_mid_body, relu_first),
        grid=(N_GRID,),
        in_specs=[
            pl.BlockSpec((NC, ROW_BLK, D), lambda i: (0, i, 0)),
            pl.BlockSpec((NC, ROW_BLK, 1), lambda i: (0, i, 0)),
            _full_spec((1, D)),
            _full_spec((D, D)),
        ],
        out_specs=_row_spec(ROW_BLK, D),
        out_shape=jax.ShapeDtypeStruct((N_PAD, D), jnp.float32),
    )(S, deg2, b, W)


def _tc4(GA, GB, Wl2, bl2):
    return pl.pallas_call(
        _tc4_body,
        grid=(P_GRID,),
        in_specs=[
            _row_spec(PAIR_BLK, D),
            _row_spec(PAIR_BLK, D),
            _full_spec((D, 1)),
            _full_spec((1, 1)),
        ],
        out_specs=_row_spec(PAIR_BLK, 1),
        out_shape=jax.ShapeDtypeStruct((P_PAD, 1), jnp.float32),
    )(GA, GB, Wl2, bl2)


def kernel(x, edge_index, node_pairs, W1, b1, W2, b2, Wl1, bl1, Wl2, bl2):
    x_pad = jnp.pad(x, ((0, N_PAD - N), (0, 0)))
    npad_e = E_PAD - E
    src = jnp.pad(edge_index[0], (0, npad_e))
    dst = jnp.concatenate(
        [edge_index[1], N + (jnp.arange(npad_e, dtype=jnp.int32) % (N_PAD - N))])
    sidx = jnp.pad(node_pairs[:, 0], (0, P_PAD - P))
    tidx = jnp.pad(node_pairs[:, 1], (0, P_PAD - P))

    zeros_row = jnp.zeros((N_PAD,), jnp.float32)
    zeros_big = jnp.zeros((N_PAD, D), jnp.float32)

    deg2 = _sc_degree(dst, zeros_row).reshape(NC, N_PAD, 1)

    Wl1_top = Wl1[:D]
    Wl1_bot = Wl1[D:]
    h1p, Bpre = _tc1(x_pad, W1, Wl1_bot, bl1.reshape(1, D), deg2)

    S1 = _sc_message_pass(h1p, src, dst, zeros_big)
    H2p = _tc_mid(True, S1, deg2, b1.reshape(1, D), W2)

    S2 = _sc_message_pass(H2p, src, dst, zeros_big)
    Apre = _tc_mid(False, S2, deg2, b2.reshape(1, D), Wl1_top)

    GA, GB = _sc_pair_gather(Apre, Bpre, sidx, tidx)
    out = _tc4(GA, GB, Wl2, bl2.reshape(1, 1))
    return out[:P]


# speedup vs baseline: 2.0418x; 2.0418x over previous
import functools

import jax
import jax.numpy as jnp
from jax import lax
from jax.experimental import pallas as pl
from jax.experimental.pallas import tpu as pltpu, tpu_sc as plsc

N = 10000
E = 320000
D = 128
P = 100000

N_PAD = 10240
P_PAD = 102400

NC, NS = 2, 16
NW = NC * NS

E_PER_W = E // NW
E_CHUNK = 200
E_STEPS = E_PER_W // E_CHUNK

DEG_CHUNK = 200
DEG_STEPS = E_PER_W // DEG_CHUNK

P_PER_W = P_PAD // NW
P_CHUNK = 320
P_STEPS = P_PER_W // P_CHUNK

ROWS_PER_SUB = N_PAD // NS

_mesh = plsc.VectorSubcoreMesh(core_axis_name="c", subcore_axis_name="s")


@functools.partial(
    pl.kernel,
    out_type=jax.ShapeDtypeStruct((NC, N_PAD), jnp.float32),
    mesh=_mesh,
    scratch_types=[
        pltpu.VMEM((E_PER_W,), jnp.int32),
        pltpu.VMEM((DEG_CHUNK,), jnp.float32),
        pltpu.VMEM_SHARED((N_PAD,), jnp.float32),
    ],
)
def _sc_degree(dst_hbm, zeros_hbm, out_hbm, didx_v, ones_v, acc_sh):
    cid = lax.axis_index("c")
    sid = lax.axis_index("s")
    wid = sid * NC + cid

    def fill(i, carry):
        ones_v[pl.ds(i * 16, 16)] = jnp.full((16,), 1.0, jnp.float32)
        return carry
    lax.fori_loop(0, DEG_CHUNK // 16, fill, None)

    e0 = pl.multiple_of(wid * E_PER_W, 8)
    pltpu.sync_copy(dst_hbm.at[pl.ds(e0, E_PER_W)], didx_v)

    row0 = sid * ROWS_PER_SUB
    pltpu.sync_copy(zeros_hbm.at[pl.ds(row0, ROWS_PER_SUB)],
                    acc_sh.at[pl.ds(row0, ROWS_PER_SUB)])
    plsc.subcore_barrier()

    def step(j, carry):
        idx = didx_v.at[pl.ds(j * DEG_CHUNK, DEG_CHUNK)]
        pltpu.sync_copy(ones_v, acc_sh.at[idx], add=True)
        return carry
    lax.fori_loop(0, DEG_STEPS, step, None)

    plsc.subcore_barrier()
    pltpu.sync_copy(acc_sh.at[pl.ds(row0, ROWS_PER_SUB)],
                    out_hbm.at[cid, pl.ds(row0, ROWS_PER_SUB)])


@functools.partial(
    pl.kernel,
    out_type=jax.ShapeDtypeStruct((NC, N_PAD, D), jnp.float32),
    mesh=_mesh,
    scratch_types=[
        pltpu.VMEM((E_CHUNK,), jnp.int32),
        pltpu.VMEM((E_CHUNK,), jnp.int32),
        pltpu.VMEM((E_CHUNK, D), jnp.float32),
        pltpu.VMEM_SHARED((N_PAD, D), jnp.float32),
        pltpu.SemaphoreType.DMA,
    ],
)
def _sc_message_pass(hp_hbm, src_hbm, dst_hbm, zeros_hbm, out_hbm,
                     sidx_v, didx_v, rows0, acc_sh, sem0):
    cid = lax.axis_index("c")
    sid = lax.axis_index("s")
    wid = sid * NC + cid

    row0 = sid * ROWS_PER_SUB
    @pl.when(cid == 0)
    def _():
        pltpu.sync_copy(hp_hbm.at[pl.ds(row0, ROWS_PER_SUB)],
                        acc_sh.at[pl.ds(row0, ROWS_PER_SUB)])
    @pl.when(cid != 0)
    def _():
        pltpu.sync_copy(zeros_hbm.at[pl.ds(row0, ROWS_PER_SUB)],
                        acc_sh.at[pl.ds(row0, ROWS_PER_SUB)])
    plsc.subcore_barrier()

    def step(t, carry):
        e0 = pl.multiple_of(wid * E_PER_W + t * E_CHUNK, 8)
        pltpu.sync_copy(src_hbm.at[pl.ds(e0, E_CHUNK)], sidx_v)
        pltpu.sync_copy(dst_hbm.at[pl.ds(e0, E_CHUNK)], didx_v)
        pltpu.async_copy(hp_hbm.at[sidx_v], rows0, sem0).wait()
        pltpu.sync_copy(rows0, acc_sh.at[didx_v], add=True)
        return carry
    lax.fori_loop(0, E_STEPS, step, None)

    plsc.subcore_barrier()
    pltpu.sync_copy(acc_sh.at[pl.ds(row0, ROWS_PER_SUB)],
                    out_hbm.at[cid, pl.ds(row0, ROWS_PER_SUB)])


@functools.partial(
    pl.kernel,
    out_type=(jax.ShapeDtypeStruct((P_PAD, D), jnp.float32),
              jax.ShapeDtypeStruct((P_PAD, D), jnp.float32)),
    mesh=_mesh,
    scratch_types=[
        pltpu.VMEM((P_CHUNK,), jnp.int32),
        pltpu.VMEM((P_CHUNK,), jnp.int32),
        pltpu.VMEM((P_CHUNK, D), jnp.float32),
        pltpu.VMEM((P_CHUNK, D), jnp.float32),
        pltpu.SemaphoreType.DMA,
    ],
)
def _sc_pair_gather(a_hbm, b_hbm, sidx_hbm, tidx_hbm, ga_hbm, gb_hbm,
                    si_v, ti_v, a0, b0, sem0):
    cid = lax.axis_index("c")
    sid = lax.axis_index("s")
    wid = sid * NC + cid

    def step(t, carry):
        p0 = pl.multiple_of(wid * P_PER_W + t * P_CHUNK, 8)
        pltpu.sync_copy(sidx_hbm.at[pl.ds(p0, P_CHUNK)], si_v)
        pltpu.sync_copy(tidx_hbm.at[pl.ds(p0, P_CHUNK)], ti_v)
        cpa = pltpu.async_copy(a_hbm.at[si_v], a0, sem0)
        cpb = pltpu.async_copy(b_hbm.at[ti_v], b0, sem0)
        cpa.wait()
        cpb.wait()
        pltpu.sync_copy(a0, ga_hbm.at[pl.ds(p0, P_CHUNK)])
        pltpu.sync_copy(b0, gb_hbm.at[pl.ds(p0, P_CHUNK)])
        return carry
    lax.fori_loop(0, P_STEPS, step, None)


ROW_BLK = 512
N_GRID = N_PAD // ROW_BLK

_HIGH = jax.lax.Precision.HIGHEST


def _dinv_from(deg_ref):
    deg = deg_ref[0] + deg_ref[1] + 1.0
    return lax.rsqrt(deg)


def _tc1_body(x_ref, w1_ref, wlb_ref, bl1_ref, deg_ref, h1p_ref, bpre_ref):
    x = x_ref[...]
    dinv = _dinv_from(deg_ref)
    h = jnp.dot(x, w1_ref[...], precision=_HIGH, preferred_element_type=jnp.float32)
    h1p_ref[...] = h * dinv
    bpre_ref[...] = jnp.dot(x, wlb_ref[...], precision=_HIGH,
                            preferred_element_type=jnp.float32) + bl1_ref[...]


def _tc_mid_body(relu_first, s_ref, deg_ref, b_ref, w_ref, out_ref):
    dinv = _dinv_from(deg_ref)
    s = s_ref[0] + s_ref[1]
    h = dinv * s + b_ref[...]
    if relu_first:
        h = jnp.maximum(h, 0.0)
        out_ref[...] = jnp.dot(h, w_ref[...], precision=_HIGH,
                               preferred_element_type=jnp.float32) * dinv
    else:
        out_ref[...] = jnp.dot(h, w_ref[...], precision=_HIGH,
                               preferred_element_type=jnp.float32)


PAIR_BLK = 2048
P_GRID = P_PAD // PAIR_BLK


def _tc4_body(ga_ref, gb_ref, wl2_ref, bl2_ref, out_ref):
    z1 = jnp.maximum(ga_ref[...] + gb_ref[...], 0.0)
    z = jnp.dot(z1, wl2_ref[...], precision=_HIGH,
                preferred_element_type=jnp.float32) + bl2_ref[...]
    out_ref[...] = jax.nn.sigmoid(z)


def _row_spec(blk, cols):
    return pl.BlockSpec((blk, cols), lambda i: (i, 0))


def _full_spec(shape):
    return pl.BlockSpec(shape, lambda i: tuple(0 for _ in shape))


def _tc1(x_pad, W1, Wlb, bl1, deg2):
    return pl.pallas_call(
        _tc1_body,
        grid=(N_GRID,),
        in_specs=[
            _row_spec(ROW_BLK, D),
            _full_spec((D, D)),
            _full_spec((D, D)),
            _full_spec((1, D)),
            pl.BlockSpec((NC, ROW_BLK, 1), lambda i: (0, i, 0)),
        ],
        out_specs=[_row_spec(ROW_BLK, D), _row_spec(ROW_BLK, D)],
        out_shape=[jax.ShapeDtypeStruct((N_PAD, D), jnp.float32),
                   jax.ShapeDtypeStruct((N_PAD, D), jnp.float32)],
    )(x_pad, W1, Wlb, bl1, deg2)


def _tc_mid(relu_first, S, deg2, b, W):
    return pl.pallas_call(
        functools.partial(_tc_mid_body, relu_first),
        grid=(N_GRID,),
        in_specs=[
            pl.BlockSpec((NC, ROW_BLK, D), lambda i: (0, i, 0)),
            pl.BlockSpec((NC, ROW_BLK, 1), lambda i: (0, i, 0)),
            _full_spec((1, D)),
            _full_spec((D, D)),
        ],
        out_specs=_row_spec(ROW_BLK, D),
        out_shape=jax.ShapeDtypeStruct((N_PAD, D), jnp.float32),
    )(S, deg2, b, W)


def _tc4(GA, GB, Wl2, bl2):
    return pl.pallas_call(
        _tc4_body,
        grid=(P_GRID,),
        in_specs=[
            _row_spec(PAIR_BLK, D),
            _row_spec(PAIR_BLK, D),
            _full_spec((D, 1)),
            _full_spec((1, 1)),
        ],
        out_specs=_row_spec(PAIR_BLK, 1),
        out_shape=jax.ShapeDtypeStruct((P_PAD, 1), jnp.float32),
    )(GA, GB, Wl2, bl2)


def kernel(x, edge_index, node_pairs, W1, b1, W2, b2, Wl1, bl1, Wl2, bl2):
    x_pad = jnp.pad(x, ((0, N_PAD - N), (0, 0)))
    src = edge_index[0]
    dst = edge_index[1]
    pad_idx = jnp.arange(P_PAD - P, dtype=jnp.int32) % N
    sidx = jnp.concatenate([node_pairs[:, 0], pad_idx])
    tidx = jnp.concatenate([node_pairs[:, 1], pad_idx])

    zeros_row = jnp.zeros((N_PAD,), jnp.float32)
    zeros_big = jnp.zeros((N_PAD, D), jnp.float32)

    deg2 = _sc_degree(dst, zeros_row).reshape(NC, N_PAD, 1)

    Wl1_top = Wl1[:D]
    Wl1_bot = Wl1[D:]
    h1p, Bpre = _tc1(x_pad, W1, Wl1_bot, bl1.reshape(1, D), deg2)

    S1 = _sc_message_pass(h1p, src, dst, zeros_big)
    H2p = _tc_mid(True, S1, deg2, b1.reshape(1, D), W2)

    S2 = _sc_message_pass(H2p, src, dst, zeros_big)
    Apre = _tc_mid(False, S2, deg2, b2.reshape(1, D), Wl1_top)

    GA, GB = _sc_pair_gather(Apre, Bpre, sidx, tidx)
    out = _tc4(GA, GB, Wl2, bl2.reshape(1, 1))
    return out[:P]


# speedup vs baseline: 2.6024x; 1.2745x over previous
import functools

import jax
import jax.numpy as jnp
from jax import lax
from jax.experimental import pallas as pl
from jax.experimental.pallas import tpu as pltpu, tpu_sc as plsc

N = 10000
E = 320000
D = 128
P = 100000

N_PAD = 10240
P_PAD = 102400

NC, NS = 2, 16
NW = NC * NS

E_PER_W = E // NW
E_CHUNK = 80
E_STEPS = E_PER_W // E_CHUNK
E_PAIRS = E_STEPS // 2
N_ACC = N_PAD
ACC_PER_SUB = N_ACC // NS

DEG_CHUNK = 200
DEG_STEPS = E_PER_W // DEG_CHUNK

P_PER_W = P_PAD // NW
P_CHUNK = 320
P_STEPS = P_PER_W // P_CHUNK

ROWS_PER_SUB = N_PAD // NS

_mesh = plsc.VectorSubcoreMesh(core_axis_name="c", subcore_axis_name="s")


@functools.partial(
    pl.kernel,
    out_type=jax.ShapeDtypeStruct((NC, N_PAD), jnp.float32),
    mesh=_mesh,
    scratch_types=[
        pltpu.VMEM((E_PER_W,), jnp.int32),
        pltpu.VMEM((DEG_CHUNK,), jnp.float32),
        pltpu.VMEM_SHARED((N_PAD,), jnp.float32),
    ],
)
def _sc_degree(dst_hbm, zeros_hbm, out_hbm, didx_v, ones_v, acc_sh):
    cid = lax.axis_index("c")
    sid = lax.axis_index("s")
    wid = sid * NC + cid

    def fill(i, carry):
        ones_v[pl.ds(i * 16, 16)] = jnp.full((16,), 1.0, jnp.float32)
        return carry
    lax.fori_loop(0, DEG_CHUNK // 16, fill, None)

    e0 = pl.multiple_of(wid * E_PER_W, 8)
    pltpu.sync_copy(dst_hbm.at[pl.ds(e0, E_PER_W)], didx_v)

    row0 = sid * ROWS_PER_SUB
    pltpu.sync_copy(zeros_hbm.at[pl.ds(row0, ROWS_PER_SUB)],
                    acc_sh.at[pl.ds(row0, ROWS_PER_SUB)])
    plsc.subcore_barrier()

    def step(j, carry):
        idx = didx_v.at[pl.ds(j * DEG_CHUNK, DEG_CHUNK)]
        pltpu.sync_copy(ones_v, acc_sh.at[idx], add=True)
        return carry
    lax.fori_loop(0, DEG_STEPS, step, None)

    plsc.subcore_barrier()
    pltpu.sync_copy(acc_sh.at[pl.ds(row0, ROWS_PER_SUB)],
                    out_hbm.at[cid, pl.ds(row0, ROWS_PER_SUB)])


@functools.partial(
    pl.kernel,
    out_type=jax.ShapeDtypeStruct((NC, N_PAD, D), jnp.float32),
    mesh=_mesh,
    scratch_types=[
        pltpu.VMEM((E_PER_W,), jnp.int32),
        pltpu.VMEM((E_PER_W,), jnp.int32),
        pltpu.VMEM((E_CHUNK, D), jnp.float32),
        pltpu.VMEM((E_CHUNK, D), jnp.float32),
        pltpu.VMEM_SHARED((N_ACC, D), jnp.float32),
        pltpu.SemaphoreType.DMA,
        pltpu.SemaphoreType.DMA,
    ],
)
def _sc_message_pass(hp_hbm, src_hbm, dst_hbm, zeros_hbm, out_hbm,
                     sidx_v, didx_v, rows0, rows1, acc_sh, sem0, sem1):
    cid = lax.axis_index("c")
    sid = lax.axis_index("s")
    wid = sid * NC + cid

    e0 = pl.multiple_of(wid * E_PER_W, 8)
    pltpu.sync_copy(src_hbm.at[pl.ds(e0, E_PER_W)], sidx_v)
    pltpu.sync_copy(dst_hbm.at[pl.ds(e0, E_PER_W)], didx_v)

    row0 = sid * ACC_PER_SUB
    @pl.when(cid == 0)
    def _():
        pltpu.sync_copy(hp_hbm.at[pl.ds(row0, ACC_PER_SUB)],
                        acc_sh.at[pl.ds(row0, ACC_PER_SUB)])
    @pl.when(cid != 0)
    def _():
        pltpu.sync_copy(zeros_hbm.at[pl.ds(row0, ACC_PER_SUB)],
                        acc_sh.at[pl.ds(row0, ACC_PER_SUB)])
    plsc.subcore_barrier()

    def sidx_of(c):
        return sidx_v.at[pl.ds(c * E_CHUNK, E_CHUNK)]

    def didx_of(c):
        return didx_v.at[pl.ds(c * E_CHUNK, E_CHUNK)]

    pltpu.async_copy(hp_hbm.at[sidx_of(0)], rows0, sem0)

    def step(t, carry):
        pltpu.async_copy(hp_hbm.at[sidx_of(2 * t + 1)], rows1, sem1)
        pltpu.make_async_copy(hp_hbm.at[sidx_of(2 * t)], rows0, sem0).wait()
        pltpu.sync_copy(rows0, acc_sh.at[didx_of(2 * t)], add=True)

        pltpu.async_copy(hp_hbm.at[sidx_of(2 * t + 2)], rows0, sem0)
        pltpu.make_async_copy(hp_hbm.at[sidx_of(2 * t + 1)], rows1, sem1).wait()
        pltpu.sync_copy(rows1, acc_sh.at[didx_of(2 * t + 1)], add=True)
        return carry
    lax.fori_loop(0, E_PAIRS, step, None)

    pltpu.make_async_copy(hp_hbm.at[sidx_of(E_STEPS - 1)], rows0, sem0).wait()
    pltpu.sync_copy(rows0, acc_sh.at[didx_of(E_STEPS - 1)], add=True)

    plsc.subcore_barrier()
    pltpu.sync_copy(acc_sh.at[pl.ds(row0, ACC_PER_SUB)],
                    out_hbm.at[cid, pl.ds(row0, ACC_PER_SUB)])


@functools.partial(
    pl.kernel,
    out_type=(jax.ShapeDtypeStruct((P_PAD, D), jnp.float32),
              jax.ShapeDtypeStruct((P_PAD, D), jnp.float32)),
    mesh=_mesh,
    scratch_types=[
        pltpu.VMEM((P_CHUNK,), jnp.int32),
        pltpu.VMEM((P_CHUNK,), jnp.int32),
        pltpu.VMEM((P_CHUNK, D), jnp.float32),
        pltpu.VMEM((P_CHUNK, D), jnp.float32),
        pltpu.SemaphoreType.DMA,
    ],
)
def _sc_pair_gather(a_hbm, b_hbm, sidx_hbm, tidx_hbm, ga_hbm, gb_hbm,
                    si_v, ti_v, a0, b0, sem0):
    cid = lax.axis_index("c")
    sid = lax.axis_index("s")
    wid = sid * NC + cid

    def step(t, carry):
        p0 = pl.multiple_of(wid * P_PER_W + t * P_CHUNK, 8)
        pltpu.sync_copy(sidx_hbm.at[pl.ds(p0, P_CHUNK)], si_v)
        pltpu.sync_copy(tidx_hbm.at[pl.ds(p0, P_CHUNK)], ti_v)
        cpa = pltpu.async_copy(a_hbm.at[si_v], a0, sem0)
        cpb = pltpu.async_copy(b_hbm.at[ti_v], b0, sem0)
        cpa.wait()
        cpb.wait()
        pltpu.sync_copy(a0, ga_hbm.at[pl.ds(p0, P_CHUNK)])
        pltpu.sync_copy(b0, gb_hbm.at[pl.ds(p0, P_CHUNK)])
        return carry
    lax.fori_loop(0, P_STEPS, step, None)


ROW_BLK = 512
N_GRID = N_PAD // ROW_BLK

_HIGH = jax.lax.Precision.HIGHEST


def _dinv_from(deg_ref):
    deg = deg_ref[0] + deg_ref[1] + 1.0
    return lax.rsqrt(deg)


def _tc1_body(x_ref, w1_ref, wlb_ref, bl1_ref, deg_ref, h1p_ref, bpre_ref):
    x = x_ref[...]
    dinv = _dinv_from(deg_ref)
    h = jnp.dot(x, w1_ref[...], precision=_HIGH, preferred_element_type=jnp.float32)
    h1p_ref[...] = h * dinv
    bpre_ref[...] = jnp.dot(x, wlb_ref[...], precision=_HIGH,
                            preferred_element_type=jnp.float32) + bl1_ref[...]


def _tc_mid_body(relu_first, s_ref, deg_ref, b_ref, w_ref, out_ref):
    dinv = _dinv_from(deg_ref)
    s = s_ref[0] + s_ref[1]
    h = dinv * s + b_ref[...]
    if relu_first:
        h = jnp.maximum(h, 0.0)
        out_ref[...] = jnp.dot(h, w_ref[...], precision=_HIGH,
                               preferred_element_type=jnp.float32) * dinv
    else:
        out_ref[...] = jnp.dot(h, w_ref[...], precision=_HIGH,
                               preferred_element_type=jnp.float32)


PAIR_BLK = 2048
P_GRID = P_PAD // PAIR_BLK


def _tc4_body(ga_ref, gb_ref, wl2_ref, bl2_ref, out_ref):
    z1 = jnp.maximum(ga_ref[...] + gb_ref[...], 0.0)
    z = jnp.dot(z1, wl2_ref[...], precision=_HIGH,
                preferred_element_type=jnp.float32) + bl2_ref[...]
    out_ref[...] = jax.nn.sigmoid(z)


def _row_spec(blk, cols):
    return pl.BlockSpec((blk, cols), lambda i: (i, 0))


def _full_spec(shape):
    return pl.BlockSpec(shape, lambda i: tuple(0 for _ in shape))


def _tc1(x_pad, W1, Wlb, bl1, deg2):
    return pl.pallas_call(
        _tc1_body,
        grid=(N_GRID,),
        in_specs=[
            _row_spec(ROW_BLK, D),
            _full_spec((D, D)),
            _full_spec((D, D)),
            _full_spec((1, D)),
            pl.BlockSpec((NC, ROW_BLK, 1), lambda i: (0, i, 0)),
        ],
        out_specs=[_row_spec(ROW_BLK, D), _row_spec(ROW_BLK, D)],
        out_shape=[jax.ShapeDtypeStruct((N_PAD, D), jnp.float32),
                   jax.ShapeDtypeStruct((N_PAD, D), jnp.float32)],
    )(x_pad, W1, Wlb, bl1, deg2)


def _tc_mid(relu_first, S, deg2, b, W):
    return pl.pallas_call(
        functools.partial(_tc_mid_body, relu_first),
        grid=(N_GRID,),
        in_specs=[
            pl.BlockSpec((NC, ROW_BLK, D), lambda i: (0, i, 0)),
            pl.BlockSpec((NC, ROW_BLK, 1), lambda i: (0, i, 0)),
            _full_spec((1, D)),
            _full_spec((D, D)),
        ],
        out_specs=_row_spec(ROW_BLK, D),
        out_shape=jax.ShapeDtypeStruct((N_PAD, D), jnp.float32),
    )(S, deg2, b, W)


def _tc4(GA, GB, Wl2, bl2):
    return pl.pallas_call(
        _tc4_body,
        grid=(P_GRID,),
        in_specs=[
            _row_spec(PAIR_BLK, D),
            _row_spec(PAIR_BLK, D),
            _full_spec((D, 1)),
            _full_spec((1, 1)),
        ],
        out_specs=_row_spec(PAIR_BLK, 1),
        out_shape=jax.ShapeDtypeStruct((P_PAD, 1), jnp.float32),
    )(GA, GB, Wl2, bl2)


def kernel(x, edge_index, node_pairs, W1, b1, W2, b2, Wl1, bl1, Wl2, bl2):
    x_pad = jnp.pad(x, ((0, N_PAD - N), (0, 0)))
    src = edge_index[0]
    dst = edge_index[1]
    pad_idx = jnp.arange(P_PAD - P, dtype=jnp.int32) % N
    sidx = jnp.concatenate([node_pairs[:, 0], pad_idx])
    tidx = jnp.concatenate([node_pairs[:, 1], pad_idx])

    zeros_row = jnp.zeros((N_PAD,), jnp.float32)
    zeros_big = jnp.zeros((N_PAD, D), jnp.float32)

    deg2 = _sc_degree(dst, zeros_row).reshape(NC, N_PAD, 1)

    Wl1_top = Wl1[:D]
    Wl1_bot = Wl1[D:]
    h1p, Bpre = _tc1(x_pad, W1, Wl1_bot, bl1.reshape(1, D), deg2)

    S1 = _sc_message_pass(h1p, src, dst, zeros_big)
    H2p = _tc_mid(True, S1, deg2, b1.reshape(1, D), W2)

    S2 = _sc_message_pass(H2p, src, dst, zeros_big)
    Apre = _tc_mid(False, S2, deg2, b2.reshape(1, D), Wl1_top)

    GA, GB = _sc_pair_gather(Apre, Bpre, sidx, tidx)
    out = _tc4(GA, GB, Wl2, bl2.reshape(1, 1))
    return out[:P]


# speedup vs baseline: 2.8022x; 1.0768x over previous
import functools

import jax
import jax.numpy as jnp
from jax import lax
from jax.experimental import pallas as pl
from jax.experimental.pallas import tpu as pltpu, tpu_sc as plsc

N = 10000
E = 320000
D = 128
P = 100000

N_PAD = 10240
P_PAD = 102400

NC, NS = 2, 16
NW = NC * NS

E_PER_W = E // NW
E_CHUNK = 80
E_STEPS = E_PER_W // E_CHUNK
E_PAIRS = E_STEPS // 2
N_ACC = N_PAD
ACC_PER_SUB = N_ACC // NS

DEG_CHUNK = 200
DEG_STEPS = E_PER_W // DEG_CHUNK

P_PER_W = P_PAD // NW
P_CHUNK = 200
P_STEPS = P_PER_W // P_CHUNK
P_PAIRS = P_STEPS // 2

ROWS_PER_SUB = N_PAD // NS

_mesh = plsc.VectorSubcoreMesh(core_axis_name="c", subcore_axis_name="s")


@functools.partial(
    pl.kernel,
    out_type=jax.ShapeDtypeStruct((NC, N_PAD), jnp.float32),
    mesh=_mesh,
    scratch_types=[
        pltpu.VMEM((E_PER_W,), jnp.int32),
        pltpu.VMEM((DEG_CHUNK,), jnp.float32),
        pltpu.VMEM_SHARED((N_PAD,), jnp.float32),
    ],
)
def _sc_degree(dst_hbm, zeros_hbm, out_hbm, didx_v, ones_v, acc_sh):
    cid = lax.axis_index("c")
    sid = lax.axis_index("s")
    wid = sid * NC + cid

    def fill(i, carry):
        ones_v[pl.ds(i * 16, 16)] = jnp.full((16,), 1.0, jnp.float32)
        return carry
    lax.fori_loop(0, DEG_CHUNK // 16, fill, None)

    e0 = pl.multiple_of(wid * E_PER_W, 8)
    pltpu.sync_copy(dst_hbm.at[pl.ds(e0, E_PER_W)], didx_v)

    row0 = sid * ROWS_PER_SUB
    pltpu.sync_copy(zeros_hbm.at[pl.ds(row0, ROWS_PER_SUB)],
                    acc_sh.at[pl.ds(row0, ROWS_PER_SUB)])
    plsc.subcore_barrier()

    def step(j, carry):
        idx = didx_v.at[pl.ds(j * DEG_CHUNK, DEG_CHUNK)]
        pltpu.sync_copy(ones_v, acc_sh.at[idx], add=True)
        return carry
    lax.fori_loop(0, DEG_STEPS, step, None)

    plsc.subcore_barrier()
    pltpu.sync_copy(acc_sh.at[pl.ds(row0, ROWS_PER_SUB)],
                    out_hbm.at[cid, pl.ds(row0, ROWS_PER_SUB)])


@functools.partial(
    pl.kernel,
    out_type=jax.ShapeDtypeStruct((NC, N_PAD, D), jnp.float32),
    mesh=_mesh,
    scratch_types=[
        pltpu.VMEM((E_PER_W,), jnp.int32),
        pltpu.VMEM((E_PER_W,), jnp.int32),
        pltpu.VMEM((E_CHUNK, D), jnp.float32),
        pltpu.VMEM((E_CHUNK, D), jnp.float32),
        pltpu.VMEM_SHARED((N_ACC, D), jnp.float32),
        pltpu.SemaphoreType.DMA,
        pltpu.SemaphoreType.DMA,
    ],
)
def _sc_message_pass(hp_hbm, src_hbm, dst_hbm, zeros_hbm, out_hbm,
                     sidx_v, didx_v, rows0, rows1, acc_sh, sem0, sem1):
    cid = lax.axis_index("c")
    sid = lax.axis_index("s")
    wid = sid * NC + cid

    e0 = pl.multiple_of(wid * E_PER_W, 8)
    pltpu.sync_copy(src_hbm.at[pl.ds(e0, E_PER_W)], sidx_v)
    pltpu.sync_copy(dst_hbm.at[pl.ds(e0, E_PER_W)], didx_v)

    row0 = sid * ACC_PER_SUB
    @pl.when(cid == 0)
    def _():
        pltpu.sync_copy(hp_hbm.at[pl.ds(row0, ACC_PER_SUB)],
                        acc_sh.at[pl.ds(row0, ACC_PER_SUB)])
    @pl.when(cid != 0)
    def _():
        pltpu.sync_copy(zeros_hbm.at[pl.ds(row0, ACC_PER_SUB)],
                        acc_sh.at[pl.ds(row0, ACC_PER_SUB)])
    plsc.subcore_barrier()

    def sidx_of(c):
        return sidx_v.at[pl.ds(c * E_CHUNK, E_CHUNK)]

    def didx_of(c):
        return didx_v.at[pl.ds(c * E_CHUNK, E_CHUNK)]

    pltpu.async_copy(hp_hbm.at[sidx_of(0)], rows0, sem0)

    def step(t, carry):
        pltpu.async_copy(hp_hbm.at[sidx_of(2 * t + 1)], rows1, sem1)
        pltpu.make_async_copy(hp_hbm.at[sidx_of(2 * t)], rows0, sem0).wait()
        pltpu.sync_copy(rows0, acc_sh.at[didx_of(2 * t)], add=True)

        pltpu.async_copy(hp_hbm.at[sidx_of(2 * t + 2)], rows0, sem0)
        pltpu.make_async_copy(hp_hbm.at[sidx_of(2 * t + 1)], rows1, sem1).wait()
        pltpu.sync_copy(rows1, acc_sh.at[didx_of(2 * t + 1)], add=True)
        return carry
    lax.fori_loop(0, E_PAIRS, step, None)

    pltpu.make_async_copy(hp_hbm.at[sidx_of(E_STEPS - 1)], rows0, sem0).wait()
    pltpu.sync_copy(rows0, acc_sh.at[didx_of(E_STEPS - 1)], add=True)

    plsc.subcore_barrier()
    pltpu.sync_copy(acc_sh.at[pl.ds(row0, ACC_PER_SUB)],
                    out_hbm.at[cid, pl.ds(row0, ACC_PER_SUB)])


@functools.partial(
    pl.kernel,
    out_type=(jax.ShapeDtypeStruct((P_PAD, D), jnp.float32),
              jax.ShapeDtypeStruct((P_PAD, D), jnp.float32)),
    mesh=_mesh,
    scratch_types=[
        pltpu.VMEM((P_PER_W,), jnp.int32),
        pltpu.VMEM((P_PER_W,), jnp.int32),
        pltpu.VMEM((P_CHUNK, D), jnp.float32),
        pltpu.VMEM((P_CHUNK, D), jnp.float32),
        pltpu.VMEM((P_CHUNK, D), jnp.float32),
        pltpu.VMEM((P_CHUNK, D), jnp.float32),
        pltpu.SemaphoreType.DMA,
        pltpu.SemaphoreType.DMA,
    ],
)
def _sc_pair_gather(a_hbm, b_hbm, sidx_hbm, tidx_hbm, ga_hbm, gb_hbm,
                    si_v, ti_v, a0, b0, a1, b1, sem0, sem1):
    cid = lax.axis_index("c")
    sid = lax.axis_index("s")
    wid = sid * NC + cid

    p00 = pl.multiple_of(wid * P_PER_W, 8)
    pltpu.sync_copy(sidx_hbm.at[pl.ds(p00, P_PER_W)], si_v)
    pltpu.sync_copy(tidx_hbm.at[pl.ds(p00, P_PER_W)], ti_v)

    def start(a_v, b_v, sem, c):
        off = pl.ds(c * P_CHUNK, P_CHUNK)
        pltpu.async_copy(a_hbm.at[si_v.at[off]], a_v, sem)
        pltpu.async_copy(b_hbm.at[ti_v.at[off]], b_v, sem)

    def drain_store(a_v, b_v, sem, c):
        off = pl.ds(c * P_CHUNK, P_CHUNK)
        pltpu.make_async_copy(a_hbm.at[si_v.at[off]], a_v, sem).wait()
        pltpu.make_async_copy(b_hbm.at[ti_v.at[off]], b_v, sem).wait()
        p0 = pl.multiple_of(wid * P_PER_W + c * P_CHUNK, 8)
        pltpu.sync_copy(a_v, ga_hbm.at[pl.ds(p0, P_CHUNK)])
        pltpu.sync_copy(b_v, gb_hbm.at[pl.ds(p0, P_CHUNK)])

    start(a0, b0, sem0, 0)

    def step(t, carry):
        start(a1, b1, sem1, 2 * t + 1)
        drain_store(a0, b0, sem0, 2 * t)

        @pl.when(t + 1 < P_PAIRS)
        def _():
            start(a0, b0, sem0, 2 * t + 2)

        drain_store(a1, b1, sem1, 2 * t + 1)
        return carry
    lax.fori_loop(0, P_PAIRS, step, None)


ROW_BLK = 1024
N_GRID = N_PAD // ROW_BLK

_HIGH = jax.lax.Precision.HIGHEST


def _dinv_from(deg_ref):
    deg = deg_ref[0] + deg_ref[1] + 1.0
    return lax.rsqrt(deg)


def _tc1_body(x_ref, w1_ref, wlb_ref, bl1_ref, deg_ref, h1p_ref, bpre_ref):
    x = x_ref[...]
    dinv = _dinv_from(deg_ref)
    h = jnp.dot(x, w1_ref[...], precision=_HIGH, preferred_element_type=jnp.float32)
    h1p_ref[...] = h * dinv
    bpre_ref[...] = jnp.dot(x, wlb_ref[...], precision=_HIGH,
                            preferred_element_type=jnp.float32) + bl1_ref[...]


def _tc_mid_body(relu_first, s_ref, deg_ref, b_ref, w_ref, out_ref):
    dinv = _dinv_from(deg_ref)
    s = s_ref[0] + s_ref[1]
    h = dinv * s + b_ref[...]
    if relu_first:
        h = jnp.maximum(h, 0.0)
        out_ref[...] = jnp.dot(h, w_ref[...], precision=_HIGH,
                               preferred_element_type=jnp.float32) * dinv
    else:
        out_ref[...] = jnp.dot(h, w_ref[...], precision=_HIGH,
                               preferred_element_type=jnp.float32)


PAIR_BLK = 4096
P_GRID = P_PAD // PAIR_BLK


def _tc4_body(ga_ref, gb_ref, wl2_ref, bl2_ref, out_ref):
    z1 = jnp.maximum(ga_ref[...] + gb_ref[...], 0.0)
    z = jnp.dot(z1, wl2_ref[...], precision=_HIGH,
                preferred_element_type=jnp.float32) + bl2_ref[...]
    out_ref[...] = jax.nn.sigmoid(z)


def _row_spec(blk, cols):
    return pl.BlockSpec((blk, cols), lambda i: (i, 0))


def _full_spec(shape):
    return pl.BlockSpec(shape, lambda i: tuple(0 for _ in shape))


def _tc1(x_pad, W1, Wlb, bl1, deg2):
    return pl.pallas_call(
        _tc1_body,
        grid=(N_GRID,),
        in_specs=[
            _row_spec(ROW_BLK, D),
            _full_spec((D, D)),
            _full_spec((D, D)),
            _full_spec((1, D)),
            pl.BlockSpec((NC, ROW_BLK, 1), lambda i: (0, i, 0)),
        ],
        out_specs=[_row_spec(ROW_BLK, D), _row_spec(ROW_BLK, D)],
        out_shape=[jax.ShapeDtypeStruct((N_PAD, D), jnp.float32),
                   jax.ShapeDtypeStruct((N_PAD, D), jnp.float32)],
    )(x_pad, W1, Wlb, bl1, deg2)


def _tc_mid(relu_first, S, deg2, b, W):
    return pl.pallas_call(
        functools.partial(_tc_mid_body, relu_first),
        grid=(N_GRID,),
        in_specs=[
            pl.BlockSpec((NC, ROW_BLK, D), lambda i: (0, i, 0)),
            pl.BlockSpec((NC, ROW_BLK, 1), lambda i: (0, i, 0)),
            _full_spec((1, D)),
            _full_spec((D, D)),
        ],
        out_specs=_row_spec(ROW_BLK, D),
        out_shape=jax.ShapeDtypeStruct((N_PAD, D), jnp.float32),
    )(S, deg2, b, W)


def _tc4(GA, GB, Wl2, bl2):
    return pl.pallas_call(
        _tc4_body,
        grid=(P_GRID,),
        in_specs=[
            _row_spec(PAIR_BLK, D),
            _row_spec(PAIR_BLK, D),
            _full_spec((D, 1)),
            _full_spec((1, 1)),
        ],
        out_specs=_row_spec(PAIR_BLK, 1),
        out_shape=jax.ShapeDtypeStruct((P_PAD, 1), jnp.float32),
    )(GA, GB, Wl2, bl2)


def kernel(x, edge_index, node_pairs, W1, b1, W2, b2, Wl1, bl1, Wl2, bl2):
    x_pad = jnp.pad(x, ((0, N_PAD - N), (0, 0)))
    src = edge_index[0]
    dst = edge_index[1]
    pad_idx = jnp.arange(P_PAD - P, dtype=jnp.int32) % N
    sidx = jnp.concatenate([node_pairs[:, 0], pad_idx])
    tidx = jnp.concatenate([node_pairs[:, 1], pad_idx])

    zeros_row = jnp.zeros((N_PAD,), jnp.float32)
    zeros_big = jnp.zeros((N_PAD, D), jnp.float32)

    deg2 = _sc_degree(dst, zeros_row).reshape(NC, N_PAD, 1)

    Wl1_top = Wl1[:D]
    Wl1_bot = Wl1[D:]
    h1p, Bpre = _tc1(x_pad, W1, Wl1_bot, bl1.reshape(1, D), deg2)

    S1 = _sc_message_pass(h1p, src, dst, zeros_big)
    H2p = _tc_mid(True, S1, deg2, b1.reshape(1, D), W2)

    S2 = _sc_message_pass(H2p, src, dst, zeros_big)
    Apre = _tc_mid(False, S2, deg2, b2.reshape(1, D), Wl1_top)

    GA, GB = _sc_pair_gather(Apre, Bpre, sidx, tidx)
    out = _tc4(GA, GB, Wl2, bl2.reshape(1, 1))
    return out[:P]
